# baseline (speedup 1.0000x reference)
import functools

import jax
import jax.numpy as jnp
from jax import lax
from jax.experimental import pallas as pl
from jax.experimental.pallas import tpu as pltpu
from jax.experimental.pallas import tpu_sc as plsc

F = 26
V = 100000
D = 32
B = 16384
L = 16

NW = 32
SW = 64
NSW = B // (NW * SW)


def _sc_lookup_sum(tables_flat, xg):
    mesh = plsc.VectorSubcoreMesh(core_axis_name="c", subcore_axis_name="s")

    @functools.partial(
        pl.kernel,
        out_type=jax.ShapeDtypeStruct((B, D), jnp.float32),
        mesh=mesh,
        scratch_types=[
            pltpu.VMEM((F, SW), jnp.int32),
            pltpu.VMEM((F, SW, D), jnp.float32),
            pltpu.VMEM((SW, D), jnp.float32),
            pltpu.SemaphoreType.DMA,
        ],
        compiler_params=pltpu.CompilerParams(use_tc_tiling_on_sc=False),
    )
    def k(tab_hbm, xg_hbm, out_hbm, idx_v, rows_v, out_v, sem):
        wid = lax.axis_index("s") * 2 + lax.axis_index("c")

        @pl.loop(0, NSW)
        def _(sw):
            g = wid * NSW + sw
            pltpu.async_copy(xg_hbm.at[g], idx_v, sem).wait()

            for f in range(F):
                for c in range(SW // L):
                    sl = pl.ds(c * L, L)
                    idx_v[f, sl] = idx_v[f, sl] + f * V

            copies = [
                pltpu.async_copy(tab_hbm.at[idx_v.at[f]], rows_v.at[f], sem)
                for f in range(F)
            ]
            for cp in copies:
                cp.wait()

            @pl.loop(0, SW)
            def _(r):
                for h in range(D // L):
                    sl = pl.ds(h * L, L)
                    acc = rows_v[0, r, sl]
                    for f in range(1, F):
                        acc = acc + rows_v[f, r, sl]
                    out_v[r, sl] = acc

            pltpu.async_copy(out_v, out_hbm.at[pl.ds(g * SW, SW)], sem).wait()

    return k(tables_flat, xg)


def kernel(x, tables):
    tables_flat = tables.reshape(F * V, D)
    xg = x.astype(jnp.int32).T.reshape(F, B // SW, SW).transpose(1, 0, 2)
    return _sc_lookup_sum(tables_flat, xg)
